# Initial kernel scaffold; baseline (speedup 1.0000x reference)
import jax
import jax.numpy as jnp
from jax.experimental import pallas as pl


def kernel(px, xyz, W, b, gamma, beta):
    raise NotImplementedError("write your pallas kernel here")


# speedup vs baseline: 21.4724x; 21.4724x over previous
import jax
import jax.numpy as jnp
from jax.experimental import pallas as pl
from jax.experimental.pallas import tpu as pltpu
from jax.experimental.pallas import tpu_sc as plsc

B = 8
N = 4096
NPOINT = 1024
NSAMPLE = 16
CIN = 32
COUT = 64
CP = 128
PC = 256
GW = 128


def _fps_body(x_ref, y_ref, z_ref, cx_ref, cy_ref, cz_ref):
    x = x_ref[...]
    y = y_ref[...]
    z = z_ref[...]
    iota = jax.lax.broadcasted_iota(jnp.int32, (B, N), 1)

    lane = jax.lax.broadcasted_iota(jnp.int32, (B, 128), 1)
    zb = jnp.zeros((B, 128), dtype=jnp.float32)

    def body(j, carry):
        distance, farthest, bx, by, bz = carry
        mask = iota == farthest
        cx = jnp.sum(jnp.where(mask, x, 0.0), axis=1, keepdims=True)
        cy = jnp.sum(jnp.where(mask, y, 0.0), axis=1, keepdims=True)
        cz = jnp.sum(jnp.where(mask, z, 0.0), axis=1, keepdims=True)
        hit = lane == j
        bx = jnp.where(hit, cx, bx)
        by = jnp.where(hit, cy, by)
        bz = jnp.where(hit, cz, bz)
        dx = x - cx
        dy = y - cy
        dz = z - cz
        dist = dx * dx + dy * dy
        dist = dist + dz * dz
        distance = jnp.minimum(distance, dist)
        m = jnp.max(distance, axis=1, keepdims=True)
        farthest = jnp.min(jnp.where(distance == m, iota, N), axis=1,
                           keepdims=True)
        return distance, farthest, bx, by, bz

    d0 = jnp.full((B, N), 1e10, dtype=jnp.float32)
    f0 = jnp.zeros((B, 1), dtype=jnp.int32)
    carry = (d0, f0, zb, zb, zb)
    for c in range(NPOINT // 128):
        carry = jax.lax.fori_loop(0, 128, body, (carry[0], carry[1], zb, zb, zb))
        cx_ref[:, c * 128:(c + 1) * 128] = carry[2]
        cy_ref[:, c * 128:(c + 1) * 128] = carry[3]
        cz_ref[:, c * 128:(c + 1) * 128] = carry[4]


def _fps(x, y, z):
    out = jax.ShapeDtypeStruct((B, NPOINT), jnp.float32)
    return pl.pallas_call(
        _fps_body,
        out_shape=(out, out, out),
    )(x, y, z)


def _transform_body(p_ref, w_ref, b_ref, t_ref):
    t = jax.lax.dot_general(
        p_ref[0], w_ref[...], (((1,), (0,)), ((), ())),
        preferred_element_type=jnp.float32)
    t_ref[0] = t + b_ref[...]


def _transform(p2, wt, bias):
    return pl.pallas_call(
        _transform_body,
        grid=(B,),
        in_specs=[
            pl.BlockSpec((1, N, CIN + 3), lambda b: (b, 0, 0)),
            pl.BlockSpec((CIN + 3, CP), lambda b: (0, 0)),
            pl.BlockSpec((1, CP), lambda b: (0, 0)),
        ],
        out_specs=pl.BlockSpec((1, N, CP), lambda b: (b, 0, 0)),
        out_shape=jax.ShapeDtypeStruct((B, N, CP), jnp.float32),
    )(p2, wt, bias)


def _knn_body(c_ref, xt_ref, idx_ref):
    b = pl.program_id(0)
    c = c_ref[0]
    x3 = xt_ref[0]
    s = jax.lax.dot_general(c, x3, (((1,), (0,)), ((), ())),
                            preferred_element_type=jnp.float32)
    dist = -2.0 * s
    dist = dist + jnp.sum(c * c, axis=1, keepdims=True)
    dist = dist + jnp.sum(x3 * x3, axis=0, keepdims=True)
    dist = jnp.maximum(dist, 0.0)

    iota = jax.lax.broadcasted_iota(jnp.int32, (PC, N), 1)
    base = b * N
    for r in range(NSAMPLE):
        m = jnp.min(dist, axis=1, keepdims=True)
        am = jnp.min(jnp.where(dist == m, iota, N), axis=1, keepdims=True)
        idx_ref[0, :, r] = (am + base)[:, 0]
        dist = jnp.where(iota == am, 1e30, dist)


def _knn(new_xyz, xt):
    return pl.pallas_call(
        _knn_body,
        grid=(B, NPOINT // PC),
        in_specs=[
            pl.BlockSpec((1, PC, 3), lambda b, j: (b, j, 0)),
            pl.BlockSpec((1, 3, N), lambda b, j: (b, 0, 0)),
        ],
        out_specs=pl.BlockSpec((1, PC, NSAMPLE), lambda b, j: (b, j, 0)),
        out_shape=jax.ShapeDtypeStruct((B, NPOINT, NSAMPLE), jnp.int32),
    )(new_xyz, xt)


def _sc_gather(t2, gidx):
    total = B * NPOINT * NSAMPLE
    mesh = plsc.VectorSubcoreMesh(core_axis_name="c", subcore_axis_name="s")

    @pl.kernel(out_type=jax.ShapeDtypeStruct((total, CP), jnp.float32),
               mesh=mesh)
    def k(t_hbm, i_hbm, o_hbm):
        def body(i_vmem, o_vmem):
            pltpu.sync_copy(t_hbm.at[i_vmem.at[0]], o_vmem)

        pltpu.emit_pipeline(
            body,
            grid=(total // GW,),
            in_specs=[pl.BlockSpec((1, GW), index_map=lambda i: (0, i))],
            out_specs=[pl.BlockSpec((GW, CP), index_map=lambda i: (i, 0))],
            core_axis_name=("c", "s"),
            dimension_semantics=(pltpu.PARALLEL,),
        )(i_hbm, o_hbm)

    return k(t2, gidx)


def _stats_body(g_ref, c_ref, wt_ref, hmax_ref, hmin_ref, ps_ref, pq_ref):
    c = c_ref[0]
    w0 = wt_ref[0:1, :]
    w1 = wt_ref[1:2, :]
    w2 = wt_ref[2:3, :]
    cc = (c[:, 0:1] * w0 + c[:, 1:2] * w1) + c[:, 2:3] * w2

    h = g_ref[0, 0] - cc
    hmax = h
    hmin = h
    ps = h
    pq = h * h
    for s in range(1, NSAMPLE):
        h = g_ref[0, s] - cc
        hmax = jnp.maximum(hmax, h)
        hmin = jnp.minimum(hmin, h)
        ps = ps + h
        pq = pq + h * h
    hmax_ref[0] = hmax
    hmin_ref[0] = hmin
    ps_ref[0, 0:1, :] = jnp.sum(ps, axis=0, keepdims=True)
    pq_ref[0, 0:1, :] = jnp.sum(pq, axis=0, keepdims=True)


def _stats(g4, new_xyz, wxyz):
    big = jax.ShapeDtypeStruct((B, NPOINT, CP), jnp.float32)
    small = jax.ShapeDtypeStruct((B, 1, CP), jnp.float32)
    return pl.pallas_call(
        _stats_body,
        grid=(B,),
        in_specs=[
            pl.BlockSpec((1, NSAMPLE, NPOINT, CP), lambda b: (b, 0, 0, 0)),
            pl.BlockSpec((1, NPOINT, 3), lambda b: (b, 0, 0)),
            pl.BlockSpec((3, CP), lambda b: (0, 0)),
        ],
        out_specs=(
            pl.BlockSpec((1, NPOINT, CP), lambda b: (b, 0, 0)),
            pl.BlockSpec((1, NPOINT, CP), lambda b: (b, 0, 0)),
            pl.BlockSpec((1, 1, CP), lambda b: (b, 0, 0)),
            pl.BlockSpec((1, 1, CP), lambda b: (b, 0, 0)),
        ),
        out_shape=(big, big, small, small),
    )(g4, new_xyz, wxyz)


def _final_body(hmax_ref, hmin_ref, ps_ref, pq_ref, g_ref, bt_ref, o_ref):
    cnt = float(B * NPOINT * NSAMPLE)
    mean = jnp.sum(ps_ref[:, 0, :], axis=0, keepdims=True) / cnt
    msq = jnp.sum(pq_ref[:, 0, :], axis=0, keepdims=True) / cnt
    var = msq - mean * mean
    scale = g_ref[...] / jnp.sqrt(var + 1e-5)
    gamma = g_ref[...]
    sel = jnp.where(gamma >= 0.0, hmax_ref[0], hmin_ref[0])
    o_ref[0] = jnp.maximum((sel - mean) * scale + bt_ref[...], 0.0)


def _final(hmax, hmin, ps, pq, gamma, beta):
    return pl.pallas_call(
        _final_body,
        grid=(B,),
        in_specs=[
            pl.BlockSpec((1, NPOINT, CP), lambda b: (b, 0, 0)),
            pl.BlockSpec((1, NPOINT, CP), lambda b: (b, 0, 0)),
            pl.BlockSpec((B, 1, CP), lambda b: (0, 0, 0)),
            pl.BlockSpec((B, 1, CP), lambda b: (0, 0, 0)),
            pl.BlockSpec((1, CP), lambda b: (0, 0)),
            pl.BlockSpec((1, CP), lambda b: (0, 0)),
        ],
        out_specs=pl.BlockSpec((1, NPOINT, CP), lambda b: (b, 0, 0)),
        out_shape=jax.ShapeDtypeStruct((B, NPOINT, CP), jnp.float32),
    )(hmax, hmin, ps, pq, gamma, beta)


def kernel(px, xyz, W, b, gamma, beta):
    xt = jnp.transpose(xyz, (0, 2, 1))
    cx, cy, cz = _fps(xt[:, 0, :], xt[:, 1, :], xt[:, 2, :])
    new_xyz = jnp.stack([cx, cy, cz], axis=-1)

    p2 = jnp.concatenate([px, xyz], axis=-1)
    wt = jnp.zeros((CIN + 3, CP), jnp.float32).at[:, :COUT].set(W.T)
    bp = jnp.zeros((1, CP), jnp.float32).at[:, :COUT].set(b)
    t = _transform(p2, wt, bp)

    gidx = _knn(new_xyz, xt)
    gidx_t = jnp.transpose(gidx, (0, 2, 1)).reshape(1, B * NPOINT * NSAMPLE)

    g = _sc_gather(t.reshape(B * N, CP), gidx_t)
    g4 = g.reshape(B, NSAMPLE, NPOINT, CP)

    wxyz = jnp.zeros((3, CP), jnp.float32).at[:, :COUT].set(W[:, CIN:].T)
    gp = jnp.zeros((1, CP), jnp.float32).at[:, :COUT].set(gamma)
    bt = jnp.zeros((1, CP), jnp.float32).at[:, :COUT].set(beta)
    hmax, hmin, ps, pq = _stats(g4, new_xyz, wxyz)
    new_px = _final(hmax, hmin, ps, pq, gp, bt)[:, :, :COUT]
    return (new_px, new_xyz)


# speedup vs baseline: 23.8556x; 1.1110x over previous
import jax
import jax.numpy as jnp
from jax.experimental import pallas as pl
from jax.experimental.pallas import tpu as pltpu
from jax.experimental.pallas import tpu_sc as plsc

B = 8
N = 4096
NPOINT = 1024
NSAMPLE = 16
CIN = 32
COUT = 64
CP = 128
PC = 256
GW = 128


def _fps_body(p_ref, cx_ref, cy_ref, cz_ref):
    p = p_ref[...]
    iota = jax.lax.broadcasted_iota(jnp.int32, (3 * B, N), 1)
    lane = jax.lax.broadcasted_iota(jnp.int32, (3 * B, 128), 1)
    zb = jnp.zeros((3 * B, 128), dtype=jnp.float32)

    def body(j, carry):
        distance, f3, buf = carry
        mask = iota == f3
        c3 = jnp.sum(jnp.where(mask, p, 0.0), axis=1, keepdims=True)
        buf = jnp.where(lane == j, c3, buf)
        d = p - c3
        d = d * d
        dist = (d[0:B] + d[B:2 * B]) + d[2 * B:3 * B]
        distance = jnp.minimum(distance, dist)
        fa = jnp.argmax(distance, axis=1).astype(jnp.int32)[:, None]
        f3 = jnp.concatenate([fa, fa, fa], axis=0)
        return distance, f3, buf

    d0 = jnp.full((B, N), 1e10, dtype=jnp.float32)
    f0 = jnp.zeros((3 * B, 1), dtype=jnp.int32)
    carry = (d0, f0, zb)
    for c in range(NPOINT // 128):
        carry = jax.lax.fori_loop(0, 128, body, (carry[0], carry[1], zb))
        cx_ref[:, c * 128:(c + 1) * 128] = carry[2][0:B]
        cy_ref[:, c * 128:(c + 1) * 128] = carry[2][B:2 * B]
        cz_ref[:, c * 128:(c + 1) * 128] = carry[2][2 * B:3 * B]


def _fps(p24):
    out = jax.ShapeDtypeStruct((B, NPOINT), jnp.float32)
    return pl.pallas_call(
        _fps_body,
        out_shape=(out, out, out),
    )(p24)


def _transform_body(p_ref, w_ref, b_ref, t_ref):
    t = jax.lax.dot_general(
        p_ref[0], w_ref[...], (((1,), (0,)), ((), ())),
        preferred_element_type=jnp.float32)
    t_ref[0] = t + b_ref[...]


def _transform(p2, wt, bias):
    return pl.pallas_call(
        _transform_body,
        grid=(B,),
        in_specs=[
            pl.BlockSpec((1, N, CIN + 3), lambda b: (b, 0, 0)),
            pl.BlockSpec((CIN + 3, CP), lambda b: (0, 0)),
            pl.BlockSpec((1, CP), lambda b: (0, 0)),
        ],
        out_specs=pl.BlockSpec((1, N, CP), lambda b: (b, 0, 0)),
        out_shape=jax.ShapeDtypeStruct((B, N, CP), jnp.float32),
    )(p2, wt, bias)


def _knn_body(c_ref, xt_ref, idx_ref):
    b = pl.program_id(0)
    c = c_ref[0]
    x3 = xt_ref[0]
    s = jax.lax.dot_general(c, x3, (((1,), (0,)), ((), ())),
                            preferred_element_type=jnp.float32)
    dist = -2.0 * s
    dist = dist + jnp.sum(c * c, axis=1, keepdims=True)
    dist = dist + jnp.sum(x3 * x3, axis=0, keepdims=True)
    dist = jnp.maximum(dist, 0.0)

    iota = jax.lax.broadcasted_iota(jnp.int32, (PC, N), 1)
    base = b * N
    for r in range(NSAMPLE):
        am = jnp.argmin(dist, axis=1).astype(jnp.int32)[:, None]
        idx_ref[0, :, r] = (am + base)[:, 0]
        dist = jnp.where(iota == am, 1e30, dist)


def _knn(new_xyz, xt):
    return pl.pallas_call(
        _knn_body,
        grid=(B, NPOINT // PC),
        in_specs=[
            pl.BlockSpec((1, PC, 3), lambda b, j: (b, j, 0)),
            pl.BlockSpec((1, 3, N), lambda b, j: (b, 0, 0)),
        ],
        out_specs=pl.BlockSpec((1, PC, NSAMPLE), lambda b, j: (b, j, 0)),
        out_shape=jax.ShapeDtypeStruct((B, NPOINT, NSAMPLE), jnp.int32),
    )(new_xyz, xt)


def _sc_gather(t2, gidx):
    total = B * NPOINT * NSAMPLE
    mesh = plsc.VectorSubcoreMesh(core_axis_name="c", subcore_axis_name="s")

    @pl.kernel(out_type=jax.ShapeDtypeStruct((total, CP), jnp.float32),
               mesh=mesh)
    def k(t_hbm, i_hbm, o_hbm):
        def body(i_vmem, o_vmem):
            pltpu.sync_copy(t_hbm.at[i_vmem.at[0]], o_vmem)

        pltpu.emit_pipeline(
            body,
            grid=(total // GW,),
            in_specs=[pl.BlockSpec((1, GW), index_map=lambda i: (0, i))],
            out_specs=[pl.BlockSpec((GW, CP), index_map=lambda i: (i, 0))],
            core_axis_name=("c", "s"),
            dimension_semantics=(pltpu.PARALLEL,),
        )(i_hbm, o_hbm)

    return k(t2, gidx)


def _stats_body(g_ref, c_ref, wt_ref, hmax_ref, hmin_ref, ps_ref, pq_ref):
    c = c_ref[0]
    w0 = wt_ref[0:1, :]
    w1 = wt_ref[1:2, :]
    w2 = wt_ref[2:3, :]
    cc = (c[:, 0:1] * w0 + c[:, 1:2] * w1) + c[:, 2:3] * w2

    h = g_ref[0, 0] - cc
    hmax = h
    hmin = h
    ps = h
    pq = h * h
    for s in range(1, NSAMPLE):
        h = g_ref[0, s] - cc
        hmax = jnp.maximum(hmax, h)
        hmin = jnp.minimum(hmin, h)
        ps = ps + h
        pq = pq + h * h
    hmax_ref[0] = hmax
    hmin_ref[0] = hmin
    ps_ref[0, 0:1, :] = jnp.sum(ps, axis=0, keepdims=True)
    pq_ref[0, 0:1, :] = jnp.sum(pq, axis=0, keepdims=True)


def _stats(g4, new_xyz, wxyz):
    big = jax.ShapeDtypeStruct((B, NPOINT, CP), jnp.float32)
    small = jax.ShapeDtypeStruct((B, 1, CP), jnp.float32)
    return pl.pallas_call(
        _stats_body,
        grid=(B,),
        in_specs=[
            pl.BlockSpec((1, NSAMPLE, NPOINT, CP), lambda b: (b, 0, 0, 0)),
            pl.BlockSpec((1, NPOINT, 3), lambda b: (b, 0, 0)),
            pl.BlockSpec((3, CP), lambda b: (0, 0)),
        ],
        out_specs=(
            pl.BlockSpec((1, NPOINT, CP), lambda b: (b, 0, 0)),
            pl.BlockSpec((1, NPOINT, CP), lambda b: (b, 0, 0)),
            pl.BlockSpec((1, 1, CP), lambda b: (b, 0, 0)),
            pl.BlockSpec((1, 1, CP), lambda b: (b, 0, 0)),
        ),
        out_shape=(big, big, small, small),
    )(g4, new_xyz, wxyz)


def _final_body(hmax_ref, hmin_ref, ps_ref, pq_ref, g_ref, bt_ref, o_ref):
    cnt = float(B * NPOINT * NSAMPLE)
    mean = jnp.sum(ps_ref[:, 0, :], axis=0, keepdims=True) / cnt
    msq = jnp.sum(pq_ref[:, 0, :], axis=0, keepdims=True) / cnt
    var = msq - mean * mean
    scale = g_ref[...] / jnp.sqrt(var + 1e-5)
    gamma = g_ref[...]
    sel = jnp.where(gamma >= 0.0, hmax_ref[0], hmin_ref[0])
    o_ref[0] = jnp.maximum((sel - mean) * scale + bt_ref[...], 0.0)


def _final(hmax, hmin, ps, pq, gamma, beta):
    return pl.pallas_call(
        _final_body,
        grid=(B,),
        in_specs=[
            pl.BlockSpec((1, NPOINT, CP), lambda b: (b, 0, 0)),
            pl.BlockSpec((1, NPOINT, CP), lambda b: (b, 0, 0)),
            pl.BlockSpec((B, 1, CP), lambda b: (0, 0, 0)),
            pl.BlockSpec((B, 1, CP), lambda b: (0, 0, 0)),
            pl.BlockSpec((1, CP), lambda b: (0, 0)),
            pl.BlockSpec((1, CP), lambda b: (0, 0)),
        ],
        out_specs=pl.BlockSpec((1, NPOINT, CP), lambda b: (b, 0, 0)),
        out_shape=jax.ShapeDtypeStruct((B, NPOINT, CP), jnp.float32),
    )(hmax, hmin, ps, pq, gamma, beta)


def kernel(px, xyz, W, b, gamma, beta):
    xt = jnp.transpose(xyz, (0, 2, 1))
    p24 = jnp.concatenate([xt[:, 0, :], xt[:, 1, :], xt[:, 2, :]], axis=0)
    cx, cy, cz = _fps(p24)
    new_xyz = jnp.stack([cx, cy, cz], axis=-1)

    p2 = jnp.concatenate([px, xyz], axis=-1)
    wt = jnp.zeros((CIN + 3, CP), jnp.float32).at[:, :COUT].set(W.T)
    bp = jnp.zeros((1, CP), jnp.float32).at[:, :COUT].set(b)
    t = _transform(p2, wt, bp)

    gidx = _knn(new_xyz, xt)
    gidx_t = jnp.transpose(gidx, (0, 2, 1)).reshape(1, B * NPOINT * NSAMPLE)

    g = _sc_gather(t.reshape(B * N, CP), gidx_t)
    g4 = g.reshape(B, NSAMPLE, NPOINT, CP)

    wxyz = jnp.zeros((3, CP), jnp.float32).at[:, :COUT].set(W[:, CIN:].T)
    gp = jnp.zeros((1, CP), jnp.float32).at[:, :COUT].set(gamma)
    bt = jnp.zeros((1, CP), jnp.float32).at[:, :COUT].set(beta)
    hmax, hmin, ps, pq = _stats(g4, new_xyz, wxyz)
    new_px = _final(hmax, hmin, ps, pq, gp, bt)[:, :, :COUT]
    return (new_px, new_xyz)


# speedup vs baseline: 23.9448x; 1.0037x over previous
import jax
import jax.numpy as jnp
from jax.experimental import pallas as pl
from jax.experimental.pallas import tpu as pltpu
from jax.experimental.pallas import tpu_sc as plsc

B = 8
N = 4096
NPOINT = 1024
NSAMPLE = 16
CIN = 32
COUT = 64
CP = 128
PC = 256
GW = 128


NSTRIP = 4
SW = N // NSTRIP


def _fps_body(p_ref, cx_ref, cy_ref, cz_ref, dist_ref):
    lane = jax.lax.broadcasted_iota(jnp.int32, (3 * B, 128), 1)
    zb = jnp.zeros((3 * B, 128), dtype=jnp.float32)
    dist_ref[...] = jnp.full((B, N), 1e10, dtype=jnp.float32)

    def body(j, carry):
        f3, buf = carry
        c3 = None
        for s in range(NSTRIP):
            ps = p_ref[:, s * SW:(s + 1) * SW]
            iota_s = jax.lax.broadcasted_iota(
                jnp.int32, (3 * B, SW), 1) + s * SW
            part = jnp.sum(jnp.where(iota_s == f3, ps, 0.0), axis=1,
                           keepdims=True)
            c3 = part if c3 is None else c3 + part
        buf = jnp.where(lane == j, c3, buf)
        best_m = None
        best_a = None
        for s in range(NSTRIP):
            ps = p_ref[:, s * SW:(s + 1) * SW]
            d = ps - c3
            d = d * d
            dist = (d[0:B] + d[B:2 * B]) + d[2 * B:3 * B]
            dnew = jnp.minimum(dist_ref[:, s * SW:(s + 1) * SW], dist)
            dist_ref[:, s * SW:(s + 1) * SW] = dnew
            m = jnp.max(dnew, axis=1, keepdims=True)
            a = jnp.argmax(dnew, axis=1).astype(jnp.int32)[:, None] + s * SW
            if best_m is None:
                best_m, best_a = m, a
            else:
                take = m > best_m
                best_a = jnp.where(take, a, best_a)
                best_m = jnp.where(take, m, best_m)
        f3 = jnp.concatenate([best_a, best_a, best_a], axis=0)
        return f3, buf

    f0 = jnp.zeros((3 * B, 1), dtype=jnp.int32)
    carry = (f0, zb)
    for c in range(NPOINT // 128):
        carry = jax.lax.fori_loop(0, 128, body, (carry[0], zb))
        cx_ref[:, c * 128:(c + 1) * 128] = carry[1][0:B]
        cy_ref[:, c * 128:(c + 1) * 128] = carry[1][B:2 * B]
        cz_ref[:, c * 128:(c + 1) * 128] = carry[1][2 * B:3 * B]


def _fps(p24):
    out = jax.ShapeDtypeStruct((B, NPOINT), jnp.float32)
    return pl.pallas_call(
        _fps_body,
        out_shape=(out, out, out),
        scratch_shapes=[pltpu.VMEM((B, N), jnp.float32)],
    )(p24)


def _transform_body(p_ref, w_ref, b_ref, t_ref):
    t = jax.lax.dot_general(
        p_ref[0], w_ref[...], (((1,), (0,)), ((), ())),
        preferred_element_type=jnp.float32)
    t_ref[0] = t + b_ref[...]


def _transform(p2, wt, bias):
    return pl.pallas_call(
        _transform_body,
        grid=(B,),
        in_specs=[
            pl.BlockSpec((1, N, CIN + 3), lambda b: (b, 0, 0)),
            pl.BlockSpec((CIN + 3, CP), lambda b: (0, 0)),
            pl.BlockSpec((1, CP), lambda b: (0, 0)),
        ],
        out_specs=pl.BlockSpec((1, N, CP), lambda b: (b, 0, 0)),
        out_shape=jax.ShapeDtypeStruct((B, N, CP), jnp.float32),
    )(p2, wt, bias)


def _knn_body(c_ref, xt_ref, idx_ref):
    b = pl.program_id(0)
    c = c_ref[0]
    x3 = xt_ref[0]
    s = jax.lax.dot_general(c, x3, (((1,), (0,)), ((), ())),
                            preferred_element_type=jnp.float32)
    dist = -2.0 * s
    dist = dist + jnp.sum(c * c, axis=1, keepdims=True)
    dist = dist + jnp.sum(x3 * x3, axis=0, keepdims=True)
    dist = jnp.maximum(dist, 0.0)

    iota = jax.lax.broadcasted_iota(jnp.int32, (PC, N), 1)
    base = b * N
    for r in range(NSAMPLE):
        am = jnp.argmin(dist, axis=1).astype(jnp.int32)[:, None]
        idx_ref[0, :, r] = (am + base)[:, 0]
        dist = jnp.where(iota == am, 1e30, dist)


def _knn(new_xyz, xt):
    return pl.pallas_call(
        _knn_body,
        grid=(B, NPOINT // PC),
        in_specs=[
            pl.BlockSpec((1, PC, 3), lambda b, j: (b, j, 0)),
            pl.BlockSpec((1, 3, N), lambda b, j: (b, 0, 0)),
        ],
        out_specs=pl.BlockSpec((1, PC, NSAMPLE), lambda b, j: (b, j, 0)),
        out_shape=jax.ShapeDtypeStruct((B, NPOINT, NSAMPLE), jnp.int32),
    )(new_xyz, xt)


def _sc_gather(t2, gidx):
    total = B * NPOINT * NSAMPLE
    mesh = plsc.VectorSubcoreMesh(core_axis_name="c", subcore_axis_name="s")

    @pl.kernel(out_type=jax.ShapeDtypeStruct((total, CP), jnp.float32),
               mesh=mesh)
    def k(t_hbm, i_hbm, o_hbm):
        def body(i_vmem, o_vmem):
            pltpu.sync_copy(t_hbm.at[i_vmem.at[0]], o_vmem)

        pltpu.emit_pipeline(
            body,
            grid=(total // GW,),
            in_specs=[pl.BlockSpec((1, GW), index_map=lambda i: (0, i))],
            out_specs=[pl.BlockSpec((GW, CP), index_map=lambda i: (i, 0))],
            core_axis_name=("c", "s"),
            dimension_semantics=(pltpu.PARALLEL,),
        )(i_hbm, o_hbm)

    return k(t2, gidx)


def _stats_body(g_ref, c_ref, wt_ref, hmax_ref, hmin_ref, ps_ref, pq_ref):
    c = c_ref[0]
    w0 = wt_ref[0:1, :]
    w1 = wt_ref[1:2, :]
    w2 = wt_ref[2:3, :]
    cc = (c[:, 0:1] * w0 + c[:, 1:2] * w1) + c[:, 2:3] * w2

    h = g_ref[0, 0] - cc
    hmax = h
    hmin = h
    ps = h
    pq = h * h
    for s in range(1, NSAMPLE):
        h = g_ref[0, s] - cc
        hmax = jnp.maximum(hmax, h)
        hmin = jnp.minimum(hmin, h)
        ps = ps + h
        pq = pq + h * h
    hmax_ref[0] = hmax
    hmin_ref[0] = hmin
    ps_ref[0, 0:1, :] = jnp.sum(ps, axis=0, keepdims=True)
    pq_ref[0, 0:1, :] = jnp.sum(pq, axis=0, keepdims=True)


def _stats(g4, new_xyz, wxyz):
    big = jax.ShapeDtypeStruct((B, NPOINT, CP), jnp.float32)
    small = jax.ShapeDtypeStruct((B, 1, CP), jnp.float32)
    return pl.pallas_call(
        _stats_body,
        grid=(B,),
        in_specs=[
            pl.BlockSpec((1, NSAMPLE, NPOINT, CP), lambda b: (b, 0, 0, 0)),
            pl.BlockSpec((1, NPOINT, 3), lambda b: (b, 0, 0)),
            pl.BlockSpec((3, CP), lambda b: (0, 0)),
        ],
        out_specs=(
            pl.BlockSpec((1, NPOINT, CP), lambda b: (b, 0, 0)),
            pl.BlockSpec((1, NPOINT, CP), lambda b: (b, 0, 0)),
            pl.BlockSpec((1, 1, CP), lambda b: (b, 0, 0)),
            pl.BlockSpec((1, 1, CP), lambda b: (b, 0, 0)),
        ),
        out_shape=(big, big, small, small),
    )(g4, new_xyz, wxyz)


def _final_body(hmax_ref, hmin_ref, ps_ref, pq_ref, g_ref, bt_ref, o_ref):
    cnt = float(B * NPOINT * NSAMPLE)
    mean = jnp.sum(ps_ref[:, 0, :], axis=0, keepdims=True) / cnt
    msq = jnp.sum(pq_ref[:, 0, :], axis=0, keepdims=True) / cnt
    var = msq - mean * mean
    scale = g_ref[...] / jnp.sqrt(var + 1e-5)
    gamma = g_ref[...]
    sel = jnp.where(gamma >= 0.0, hmax_ref[0], hmin_ref[0])
    o_ref[0] = jnp.maximum((sel - mean) * scale + bt_ref[...], 0.0)


def _final(hmax, hmin, ps, pq, gamma, beta):
    return pl.pallas_call(
        _final_body,
        grid=(B,),
        in_specs=[
            pl.BlockSpec((1, NPOINT, CP), lambda b: (b, 0, 0)),
            pl.BlockSpec((1, NPOINT, CP), lambda b: (b, 0, 0)),
            pl.BlockSpec((B, 1, CP), lambda b: (0, 0, 0)),
            pl.BlockSpec((B, 1, CP), lambda b: (0, 0, 0)),
            pl.BlockSpec((1, CP), lambda b: (0, 0)),
            pl.BlockSpec((1, CP), lambda b: (0, 0)),
        ],
        out_specs=pl.BlockSpec((1, NPOINT, CP), lambda b: (b, 0, 0)),
        out_shape=jax.ShapeDtypeStruct((B, NPOINT, CP), jnp.float32),
    )(hmax, hmin, ps, pq, gamma, beta)


def kernel(px, xyz, W, b, gamma, beta):
    xt = jnp.transpose(xyz, (0, 2, 1))
    p24 = jnp.concatenate([xt[:, 0, :], xt[:, 1, :], xt[:, 2, :]], axis=0)
    cx, cy, cz = _fps(p24)
    new_xyz = jnp.stack([cx, cy, cz], axis=-1)

    p2 = jnp.concatenate([px, xyz], axis=-1)
    wt = jnp.zeros((CIN + 3, CP), jnp.float32).at[:, :COUT].set(W.T)
    bp = jnp.zeros((1, CP), jnp.float32).at[:, :COUT].set(b)
    t = _transform(p2, wt, bp)

    gidx = _knn(new_xyz, xt)
    gidx_t = jnp.transpose(gidx, (0, 2, 1)).reshape(1, B * NPOINT * NSAMPLE)

    g = _sc_gather(t.reshape(B * N, CP), gidx_t)
    g4 = g.reshape(B, NSAMPLE, NPOINT, CP)

    wxyz = jnp.zeros((3, CP), jnp.float32).at[:, :COUT].set(W[:, CIN:].T)
    gp = jnp.zeros((1, CP), jnp.float32).at[:, :COUT].set(gamma)
    bt = jnp.zeros((1, CP), jnp.float32).at[:, :COUT].set(beta)
    hmax, hmin, ps, pq = _stats(g4, new_xyz, wxyz)
    new_px = _final(hmax, hmin, ps, pq, gp, bt)[:, :, :COUT]
    return (new_px, new_xyz)


# speedup vs baseline: 24.4390x; 1.0206x over previous
import functools

import jax
import jax.numpy as jnp
from jax.experimental import pallas as pl
from jax.experimental.pallas import tpu as pltpu
from jax.experimental.pallas import tpu_sc as plsc

B = 8
N = 4096
NPOINT = 1024
NSAMPLE = 16
CIN = 32
COUT = 64
CP = 128
HB = B // 2
PC = 256
GW = 128


NSTRIP = 4
SW = N // NSTRIP


def _fps_body(p_ref, cx_ref, cy_ref, cz_ref, dist_ref):
    lane = jax.lax.broadcasted_iota(jnp.int32, (3 * B, 128), 1)
    zb = jnp.zeros((3 * B, 128), dtype=jnp.float32)
    dist_ref[...] = jnp.full((B, N), 1e10, dtype=jnp.float32)

    def body(j, carry):
        f3, buf = carry
        c3 = None
        for s in range(NSTRIP):
            ps = p_ref[:, s * SW:(s + 1) * SW]
            iota_s = jax.lax.broadcasted_iota(
                jnp.int32, (3 * B, SW), 1) + s * SW
            part = jnp.sum(jnp.where(iota_s == f3, ps, 0.0), axis=1,
                           keepdims=True)
            c3 = part if c3 is None else c3 + part
        buf = jnp.where(lane == j, c3, buf)
        best_m = None
        best_a = None
        for s in range(NSTRIP):
            ps = p_ref[:, s * SW:(s + 1) * SW]
            d = ps - c3
            d = d * d
            dist = (d[0:B] + d[B:2 * B]) + d[2 * B:3 * B]
            dnew = jnp.minimum(dist_ref[:, s * SW:(s + 1) * SW], dist)
            dist_ref[:, s * SW:(s + 1) * SW] = dnew
            m = jnp.max(dnew, axis=1, keepdims=True)
            a = jnp.argmax(dnew, axis=1).astype(jnp.int32)[:, None] + s * SW
            if best_m is None:
                best_m, best_a = m, a
            else:
                take = m > best_m
                best_a = jnp.where(take, a, best_a)
                best_m = jnp.where(take, m, best_m)
        f3 = jnp.concatenate([best_a, best_a, best_a], axis=0)
        return f3, buf

    f0 = jnp.zeros((3 * B, 1), dtype=jnp.int32)
    carry = (f0, zb)
    for c in range(NPOINT // 128):
        carry = jax.lax.fori_loop(0, 128, body, (carry[0], zb))
        cx_ref[:, c * 128:(c + 1) * 128] = carry[1][0:B]
        cy_ref[:, c * 128:(c + 1) * 128] = carry[1][B:2 * B]
        cz_ref[:, c * 128:(c + 1) * 128] = carry[1][2 * B:3 * B]


def _fps(p24):
    out = jax.ShapeDtypeStruct((B, NPOINT), jnp.float32)
    return pl.pallas_call(
        _fps_body,
        out_shape=(out, out, out),
        scratch_shapes=[pltpu.VMEM((B, N), jnp.float32)],
    )(p24)


def _transform_body(p_ref, w_ref, b_ref, t_ref):
    t = jax.lax.dot_general(
        p_ref[0], w_ref[...], (((1,), (0,)), ((), ())),
        preferred_element_type=jnp.float32)
    t_ref[0] = t + b_ref[...]


def _transform(p2, wt, bias):
    return pl.pallas_call(
        _transform_body,
        grid=(B,),
        in_specs=[
            pl.BlockSpec((1, N, CIN + 3), lambda b: (b, 0, 0)),
            pl.BlockSpec((CIN + 3, CP), lambda b: (0, 0)),
            pl.BlockSpec((1, CP), lambda b: (0, 0)),
        ],
        out_specs=pl.BlockSpec((1, N, CP), lambda b: (b, 0, 0)),
        out_shape=jax.ShapeDtypeStruct((B, N, CP), jnp.float32),
    )(p2, wt, bias)


def _knn_body(off, c_ref, xt_ref, idx_ref):
    b = pl.program_id(0) + off
    c = c_ref[0]
    x3 = xt_ref[0]
    s = jax.lax.dot_general(c, x3, (((1,), (0,)), ((), ())),
                            preferred_element_type=jnp.float32)
    dist = -2.0 * s
    dist = dist + jnp.sum(c * c, axis=1, keepdims=True)
    dist = dist + jnp.sum(x3 * x3, axis=0, keepdims=True)
    dist = jnp.maximum(dist, 0.0)

    iota = jax.lax.broadcasted_iota(jnp.int32, (PC, N), 1)
    base = b * N
    for r in range(NSAMPLE):
        am = jnp.argmin(dist, axis=1).astype(jnp.int32)[:, None]
        idx_ref[0, :, r] = (am + base)[:, 0]
        dist = jnp.where(iota == am, 1e30, dist)


def _knn(new_xyz, xt, off):
    nb = new_xyz.shape[0]
    return pl.pallas_call(
        functools.partial(_knn_body, off),
        grid=(nb, NPOINT // PC),
        in_specs=[
            pl.BlockSpec((1, PC, 3), lambda b, j: (b, j, 0)),
            pl.BlockSpec((1, 3, N), lambda b, j: (b, 0, 0)),
        ],
        out_specs=pl.BlockSpec((1, PC, NSAMPLE), lambda b, j: (b, j, 0)),
        out_shape=jax.ShapeDtypeStruct((nb, NPOINT, NSAMPLE), jnp.int32),
    )(new_xyz, xt)


def _sc_gather(t2, gidx):
    total = gidx.shape[1]
    mesh = plsc.VectorSubcoreMesh(core_axis_name="c", subcore_axis_name="s")

    @pl.kernel(out_type=jax.ShapeDtypeStruct((total, CP), jnp.float32),
               mesh=mesh)
    def k(t_hbm, i_hbm, o_hbm):
        def body(i_vmem, o_vmem):
            pltpu.sync_copy(t_hbm.at[i_vmem.at[0]], o_vmem)

        pltpu.emit_pipeline(
            body,
            grid=(total // GW,),
            in_specs=[pl.BlockSpec((1, GW), index_map=lambda i: (0, i))],
            out_specs=[pl.BlockSpec((GW, CP), index_map=lambda i: (i, 0))],
            core_axis_name=("c", "s"),
            dimension_semantics=(pltpu.PARALLEL,),
        )(i_hbm, o_hbm)

    return k(t2, gidx)


def _stats_body(g_ref, c_ref, wt_ref, hmax_ref, hmin_ref, ps_ref, pq_ref):
    c = c_ref[0]
    w0 = wt_ref[0:1, :]
    w1 = wt_ref[1:2, :]
    w2 = wt_ref[2:3, :]
    cc = (c[:, 0:1] * w0 + c[:, 1:2] * w1) + c[:, 2:3] * w2

    h = g_ref[0, 0] - cc
    hmax = h
    hmin = h
    ps = h
    pq = h * h
    for s in range(1, NSAMPLE):
        h = g_ref[0, s] - cc
        hmax = jnp.maximum(hmax, h)
        hmin = jnp.minimum(hmin, h)
        ps = ps + h
        pq = pq + h * h
    hmax_ref[0] = hmax
    hmin_ref[0] = hmin
    ps_ref[0, 0:1, :] = jnp.sum(ps, axis=0, keepdims=True)
    pq_ref[0, 0:1, :] = jnp.sum(pq, axis=0, keepdims=True)


def _stats(g4, new_xyz, wxyz):
    nb = g4.shape[0]
    big = jax.ShapeDtypeStruct((nb, NPOINT, CP), jnp.float32)
    small = jax.ShapeDtypeStruct((nb, 1, CP), jnp.float32)
    return pl.pallas_call(
        _stats_body,
        grid=(nb,),
        in_specs=[
            pl.BlockSpec((1, NSAMPLE, NPOINT, CP), lambda b: (b, 0, 0, 0)),
            pl.BlockSpec((1, NPOINT, 3), lambda b: (b, 0, 0)),
            pl.BlockSpec((3, CP), lambda b: (0, 0)),
        ],
        out_specs=(
            pl.BlockSpec((1, NPOINT, CP), lambda b: (b, 0, 0)),
            pl.BlockSpec((1, NPOINT, CP), lambda b: (b, 0, 0)),
            pl.BlockSpec((1, 1, CP), lambda b: (b, 0, 0)),
            pl.BlockSpec((1, 1, CP), lambda b: (b, 0, 0)),
        ),
        out_shape=(big, big, small, small),
    )(g4, new_xyz, wxyz)


def _final_body(hmax_ref, hmin_ref, psa_ref, pqa_ref, psb_ref, pqb_ref,
                g_ref, bt_ref, o_ref):
    cnt = float(B * NPOINT * NSAMPLE)
    mean = (jnp.sum(psa_ref[:, 0, :], axis=0, keepdims=True)
            + jnp.sum(psb_ref[:, 0, :], axis=0, keepdims=True)) / cnt
    msq = (jnp.sum(pqa_ref[:, 0, :], axis=0, keepdims=True)
           + jnp.sum(pqb_ref[:, 0, :], axis=0, keepdims=True)) / cnt
    var = msq - mean * mean
    scale = g_ref[...] / jnp.sqrt(var + 1e-5)
    gamma = g_ref[...]
    sel = jnp.where(gamma >= 0.0, hmax_ref[0], hmin_ref[0])
    o_ref[0] = jnp.maximum((sel - mean) * scale + bt_ref[...], 0.0)


def _final(hmax, hmin, psa, pqa, psb, pqb, gamma, beta):
    nb = hmax.shape[0]
    return pl.pallas_call(
        _final_body,
        grid=(nb,),
        in_specs=[
            pl.BlockSpec((1, NPOINT, CP), lambda b: (b, 0, 0)),
            pl.BlockSpec((1, NPOINT, CP), lambda b: (b, 0, 0)),
            pl.BlockSpec((HB, 1, CP), lambda b: (0, 0, 0)),
            pl.BlockSpec((HB, 1, CP), lambda b: (0, 0, 0)),
            pl.BlockSpec((HB, 1, CP), lambda b: (0, 0, 0)),
            pl.BlockSpec((HB, 1, CP), lambda b: (0, 0, 0)),
            pl.BlockSpec((1, CP), lambda b: (0, 0)),
            pl.BlockSpec((1, CP), lambda b: (0, 0)),
        ],
        out_specs=pl.BlockSpec((1, NPOINT, CP), lambda b: (b, 0, 0)),
        out_shape=jax.ShapeDtypeStruct((nb, NPOINT, CP), jnp.float32),
    )(hmax, hmin, psa, pqa, psb, pqb, gamma, beta)


def kernel(px, xyz, W, b, gamma, beta):
    xt = jnp.transpose(xyz, (0, 2, 1))
    p24 = jnp.concatenate([xt[:, 0, :], xt[:, 1, :], xt[:, 2, :]], axis=0)
    cx, cy, cz = _fps(p24)
    new_xyz = jnp.stack([cx, cy, cz], axis=-1)

    p2 = jnp.concatenate([px, xyz], axis=-1)
    wt = jnp.zeros((CIN + 3, CP), jnp.float32).at[:, :COUT].set(W.T)
    bp = jnp.zeros((1, CP), jnp.float32).at[:, :COUT].set(b)
    t = _transform(p2, wt, bp)

    t2 = t.reshape(B * N, CP)
    idx_a = _knn(new_xyz[:HB], xt[:HB], 0)
    gia = jnp.transpose(idx_a, (0, 2, 1)).reshape(1, HB * NPOINT * NSAMPLE)
    ga = _sc_gather(t2, gia)
    idx_b = _knn(new_xyz[HB:], xt[HB:], HB)
    gib = jnp.transpose(idx_b, (0, 2, 1)).reshape(1, HB * NPOINT * NSAMPLE)
    gb = _sc_gather(t2, gib)

    wxyz = jnp.zeros((3, CP), jnp.float32).at[:, :COUT].set(W[:, CIN:].T)
    gp = jnp.zeros((1, CP), jnp.float32).at[:, :COUT].set(gamma)
    bt = jnp.zeros((1, CP), jnp.float32).at[:, :COUT].set(beta)
    hmax_a, hmin_a, ps_a, pq_a = _stats(
        ga.reshape(HB, NSAMPLE, NPOINT, CP), new_xyz[:HB], wxyz)
    hmax_b, hmin_b, ps_b, pq_b = _stats(
        gb.reshape(HB, NSAMPLE, NPOINT, CP), new_xyz[HB:], wxyz)
    out_a = _final(hmax_a, hmin_a, ps_a, pq_a, ps_b, pq_b, gp, bt)
    out_b = _final(hmax_b, hmin_b, ps_a, pq_a, ps_b, pq_b, gp, bt)
    new_px = jnp.concatenate([out_a, out_b], axis=0)[:, :, :COUT]
    return (new_px, new_xyz)
